# Initial kernel scaffold; baseline (speedup 1.0000x reference)
import jax
import jax.numpy as jnp
from jax.experimental import pallas as pl


def kernel(pixel_values_nchw, conv1_w, conv1_b, conv2_w, conv2_b, conv3_w, conv3_b, cls_w, cls_b, fc_w, fc_b):
    raise NotImplementedError("write your pallas kernel here")


# speedup vs baseline: 2.0774x; 2.0774x over previous
import functools

import jax
import jax.numpy as jnp
from jax.experimental import pallas as pl
from jax.experimental.pallas import tpu as pltpu

_C_MID = 128
_N_CLS = 10
_K_STEM = 32


def _fused_body(xc_ref, w1_ref, b1_ref, w2_ref, b2_ref, w3_ref, b3_ref,
                hw_ref, hb_ref, o_ref, pad1_ref, pad2_ref, h1_ref,
                *, H, W):
    Wp = W + 2
    M = H * Wp
    lead = 16 - Wp % 16
    edge = lead + Wp
    Mp = pad1_ref.shape[0]

    row = jax.lax.broadcasted_iota(jnp.int32, (M, 1), 0)
    col = jax.lax.rem(row, Wp)
    mask = jnp.logical_and(col >= 1, col <= W)

    def store_padded(ref, val):
        ref[0:edge, :] = jnp.zeros((edge, _C_MID), jnp.bfloat16)
        ref[edge + M:Mp, :] = jnp.zeros((Mp - edge - M, _C_MID), jnp.bfloat16)
        ref[edge:edge + M, :] = val.astype(jnp.bfloat16)

    def conv3x3(src_ref, w_ref, b_ref):
        acc = jnp.zeros((M, _C_MID), jnp.float32)
        for t in range(9):
            dh, dw = divmod(t, 3)
            s = lead + (dh - 1) * Wp + (dw - 1) + Wp
            acc = acc + jnp.dot(src_ref[s:s + M, :], w_ref[t],
                                preferred_element_type=jnp.float32)
        return acc + b_ref[...]

    h1 = jnp.dot(xc_ref[0], w1_ref[...], preferred_element_type=jnp.float32)
    h1 = jnp.maximum(h1 + b1_ref[...], 0.0)
    h1 = jnp.where(mask, h1, 0.0)
    h1_ref[...] = h1
    store_padded(pad1_ref, h1)

    r = jnp.maximum(conv3x3(pad1_ref, w2_ref, b2_ref), 0.0)
    store_padded(pad2_ref, jnp.where(mask, r, 0.0))

    h3 = conv3x3(pad2_ref, w3_ref, b3_ref) + h1_ref[...]
    h3 = jnp.where(mask, jnp.maximum(h3, 0.0), 0.0)

    pooled = jnp.sum(h3, axis=0, keepdims=True) * (1.0 / (H * W))
    logits = jnp.dot(pooled, hw_ref[...], preferred_element_type=jnp.float32)
    o_ref[...] = (logits + hb_ref[...]).reshape(1, 1, _N_CLS)


def kernel(pixel_values_nchw, conv1_w, conv1_b, conv2_w, conv2_b,
           conv3_w, conv3_b, cls_w, cls_b, fc_w, fc_b):
    N, C_in, H, W = pixel_values_nchw.shape
    Wp = W + 2
    M = H * Wp
    lead = 16 - Wp % 16
    Mp = -(-(lead + (H + 2) * Wp + 2) // 16) * 16

    x = jnp.transpose(pixel_values_nchw, (0, 2, 3, 1))
    xp = jnp.pad(x, ((0, 0), (1, 1), (1, 1), (0, 0)))
    xf = xp.reshape(N, (H + 2) * Wp, C_in)
    xf = jnp.pad(xf, ((0, 0), (1, 8), (0, 0)))
    taps = [xf[:, dh * Wp + dw:dh * Wp + dw + M, :]
            for dh in range(3) for dw in range(3)]
    xcol = jnp.concatenate(taps, axis=-1).astype(jnp.bfloat16)
    xcol = jnp.pad(xcol, ((0, 0), (0, 0), (0, _K_STEM - 9 * C_in)))

    w1 = jnp.pad(conv1_w.reshape(9 * C_in, _C_MID),
                 ((0, _K_STEM - 9 * C_in), (0, 0))).astype(jnp.bfloat16)
    w2 = conv2_w.astype(jnp.bfloat16)
    w3 = conv3_w.astype(jnp.bfloat16)
    b1 = conv1_b.reshape(1, _C_MID)
    b2 = conv2_b.reshape(1, _C_MID)
    b3 = conv3_b.reshape(1, _C_MID)

    head_w = jnp.dot(cls_w, fc_w)
    head_b = jnp.dot(cls_b[None, :], fc_w) + fc_b[None, :]

    body = functools.partial(_fused_body, H=H, W=W)

    out = pl.pallas_call(
        body,
        out_shape=jax.ShapeDtypeStruct((N, 1, _N_CLS), jnp.float32),
        grid=(N,),
        in_specs=[
            pl.BlockSpec((1, M, _K_STEM), lambda n: (n, 0, 0)),
            pl.BlockSpec((_K_STEM, _C_MID), lambda n: (0, 0)),
            pl.BlockSpec((1, _C_MID), lambda n: (0, 0)),
            pl.BlockSpec((9, _C_MID, _C_MID), lambda n: (0, 0, 0)),
            pl.BlockSpec((1, _C_MID), lambda n: (0, 0)),
            pl.BlockSpec((9, _C_MID, _C_MID), lambda n: (0, 0, 0)),
            pl.BlockSpec((1, _C_MID), lambda n: (0, 0)),
            pl.BlockSpec((_C_MID, _N_CLS), lambda n: (0, 0)),
            pl.BlockSpec((1, _N_CLS), lambda n: (0, 0)),
        ],
        out_specs=pl.BlockSpec((1, 1, _N_CLS), lambda n: (n, 0, 0)),
        scratch_shapes=[
            pltpu.VMEM((Mp, _C_MID), jnp.bfloat16),
            pltpu.VMEM((Mp, _C_MID), jnp.bfloat16),
            pltpu.VMEM((M, _C_MID), jnp.float32),
        ],
        compiler_params=pltpu.CompilerParams(
            dimension_semantics=("parallel",),
            vmem_limit_bytes=56 * 1024 * 1024,
        ),
    )(xcol, w1, b1, w2, b2, w3, b3, head_w, head_b)

    return out.reshape(N, _N_CLS)
